# Initial kernel scaffold; baseline (speedup 1.0000x reference)
import jax
import jax.numpy as jnp
from jax.experimental import pallas as pl


def kernel(enc, enc_mask, table):
    raise NotImplementedError("write your pallas kernel here")


# speedup vs baseline: 4.9427x; 4.9427x over previous
import functools

import jax
import jax.numpy as jnp
from jax import lax
from jax.experimental import pallas as pl
from jax.experimental.pallas import tpu as pltpu
from jax.experimental.pallas import tpu_sc as plsc

VOCAB = 1000000
EMBED_DIM = 32
BATCH = 16384
SEQ = 200

_B = BATCH * SEQ
_NW = 32
_BPW = _B // _NW
_C = 2048
_NCHUNK = _BPW // _C


def _gather_body(idx_hbm, table_hbm, out_hbm, idx_v, rows_v, sem):
    wid = lax.axis_index("s") * 2 + lax.axis_index("c")
    base = wid * _BPW

    def body(i, _):
        off = base + i * _C
        pltpu.sync_copy(idx_hbm.at[pl.ds(off, _C)], idx_v)
        pltpu.async_copy(table_hbm.at[idx_v], rows_v, sem).wait()
        pltpu.sync_copy(rows_v, out_hbm.at[pl.ds(off, _C)])
        return 0

    lax.fori_loop(0, _NCHUNK, body, 0)


@jax.jit
def kernel(enc, enc_mask, table):
    idx_flat = enc.reshape(_B)
    mesh = plsc.VectorSubcoreMesh(core_axis_name="c", subcore_axis_name="s")
    gather = pl.kernel(
        _gather_body,
        mesh=mesh,
        out_type=jax.ShapeDtypeStruct((_B, EMBED_DIM), jnp.float32),
        scratch_types=[
            pltpu.VMEM((_C,), jnp.int32),
            pltpu.VMEM((_C, EMBED_DIM), jnp.float32),
            pltpu.SemaphoreType.DMA,
        ],
        compiler_params=pltpu.CompilerParams(use_tc_tiling_on_sc=False),
    )
    dec = gather(idx_flat, table)
    return (dec.reshape(BATCH, SEQ, EMBED_DIM), enc_mask)


# speedup vs baseline: 5.0345x; 1.0186x over previous
import jax
import jax.numpy as jnp
from jax import lax
from jax.experimental import pallas as pl
from jax.experimental.pallas import tpu as pltpu
from jax.experimental.pallas import tpu_sc as plsc

VOCAB = 1000000
EMBED_DIM = 32
BATCH = 16384
SEQ = 200

_B = BATCH * SEQ
_NW = 32
_BPW = _B // _NW
_C = 1600
_NCHUNK = _BPW // _C


def _gather_body(idx_hbm, table_hbm, out_hbm,
                 idx0, idx1, rows0, rows1, gsem0, gsem1, osem0, osem1):
    wid = lax.axis_index("s") * 2 + lax.axis_index("c")
    base = wid * _BPW
    idx_v = (idx0, idx1)
    rows_v = (rows0, rows1)
    gsem = (gsem0, gsem1)
    osem = (osem0, osem1)

    def step(j, b):
        bp = 1 - b
        off = base + j * _C
        pltpu.sync_copy(idx_hbm.at[pl.ds(off, _C)], idx_v[b])
        @pl.when(j >= 1)
        def _wait_prev_gather():
            pltpu.make_async_copy(
                table_hbm.at[idx_v[bp]], rows_v[bp], gsem[bp]).wait()

        @pl.when(j >= 2)
        def _wait_prev_write():
            pltpu.make_async_copy(
                rows_v[b], out_hbm.at[pl.ds(off - 2 * _C, _C)], osem[b]).wait()

        pltpu.async_copy(table_hbm.at[idx_v[b]], rows_v[b], gsem[b])

        @pl.when(j >= 1)
        def _start_prev_write():
            pltpu.async_copy(
                rows_v[bp], out_hbm.at[pl.ds(off - _C, _C)], osem[bp])

    def body(j0, _):
        step(j0, 0)
        step(j0 + 1, 1)
        return 0

    lax.fori_loop(0, _NCHUNK // 2, lambda i, c: body(2 * i, c), 0)

    lastb = (_NCHUNK - 1) % 2
    last_off = base + (_NCHUNK - 1) * _C
    pltpu.make_async_copy(
        table_hbm.at[idx_v[lastb]], rows_v[lastb], gsem[lastb]).wait()
    pltpu.make_async_copy(
        rows_v[1 - lastb], out_hbm.at[pl.ds(last_off - _C, _C)],
        osem[1 - lastb]).wait()
    pltpu.sync_copy(rows_v[lastb], out_hbm.at[pl.ds(last_off, _C)])


@jax.jit
def kernel(enc, enc_mask, table):
    idx_flat = enc.reshape(_B)
    mesh = plsc.VectorSubcoreMesh(core_axis_name="c", subcore_axis_name="s")
    gather = pl.kernel(
        _gather_body,
        mesh=mesh,
        out_type=jax.ShapeDtypeStruct((_B, EMBED_DIM), jnp.float32),
        scratch_types=[
            pltpu.VMEM((_C,), jnp.int32),
            pltpu.VMEM((_C,), jnp.int32),
            pltpu.VMEM((_C, EMBED_DIM), jnp.float32),
            pltpu.VMEM((_C, EMBED_DIM), jnp.float32),
            pltpu.SemaphoreType.DMA,
            pltpu.SemaphoreType.DMA,
            pltpu.SemaphoreType.DMA,
            pltpu.SemaphoreType.DMA,
        ],
        compiler_params=pltpu.CompilerParams(use_tc_tiling_on_sc=False),
    )
    dec = gather(idx_flat, table)
    return (dec.reshape(BATCH, SEQ, EMBED_DIM), enc_mask)
